# baseline (speedup 1.0000x reference)
import functools

import jax
import jax.numpy as jnp
from jax import lax
from jax.experimental import pallas as pl
from jax.experimental.pallas import tpu as pltpu
from jax.experimental.pallas import tpu_sc as plsc

N_NODES = 10000
N_EDGES = 320000
DIM = 128

NC = 2
NS = 16
EDGES_PER_TILE = N_EDGES // (NC * NS)
CHUNK = 80
SUP = 5
SCHUNK = 25
NCHUNK = SUP * SCHUNK
LGRP = CHUNK // 16
NB = 3
LA = NB - 1
N_PAD = 10240
ROWS_PER_TILE = N_PAD // NS
ZROWS = 64
SHIFT = 14
MASK = (1 << SHIFT) - 1


def _sc_segment_sum(src_r, dst_r, entity_embed, zrows):
    mesh = plsc.VectorSubcoreMesh(core_axis_name="c", subcore_axis_name="s")

    @functools.partial(
        pl.kernel,
        mesh=mesh,
        out_type=jax.ShapeDtypeStruct((NC, N_PAD, DIM), jnp.float32),
        scratch_types=[
            pltpu.VMEM((SCHUNK, CHUNK), jnp.int32),
            pltpu.VMEM((SCHUNK, CHUNK), jnp.int32),
            pltpu.VMEM((NB, CHUNK, DIM), jnp.float32),
            pltpu.VMEM_SHARED((N_PAD, DIM), jnp.float32),
        ] + [pltpu.SemaphoreType.DMA] * (2 * NB),
    )
    def seg_sum(src_hbm, dst_hbm, ent_hbm, z_hbm, out_hbm,
                src_v, dst_v, rows_v, h_sh, *sems):
        gsem = sems[:NB]
        ssem = sems[NB:2 * NB]
        c = lax.axis_index("c")
        s = lax.axis_index("s")
        base = s * ROWS_PER_TILE
        for z in range(ROWS_PER_TILE // ZROWS):
            pltpu.async_copy(z_hbm, h_sh.at[pl.ds(base + z * ZROWS, ZROWS)],
                             gsem[0])
        for z in range(ROWS_PER_TILE // ZROWS):
            pltpu.make_async_copy(z_hbm, h_sh.at[pl.ds(base, ZROWS)],
                                  gsem[0]).wait()

        plsc.subcore_barrier()

        def gather(g, b):
            pltpu.async_copy(ent_hbm.at[src_v.at[g]], rows_v.at[b], gsem[b])

        def gather_wait(b):
            pltpu.make_async_copy(ent_hbm.at[src_v.at[0]], rows_v.at[b],
                                  gsem[b]).wait()

        def scatter(g, b):
            pltpu.async_copy(rows_v.at[b], h_sh.at[dst_v.at[g]], ssem[b],
                             add=True)

        def scatter_wait(b):
            pltpu.make_async_copy(rows_v.at[b], h_sh.at[dst_v.at[0]],
                                  ssem[b]).wait()

        def sup_body(q, carry):
            pltpu.sync_copy(src_hbm.at[c, s, q], src_v)
            pltpu.sync_copy(dst_hbm.at[c, s, q], dst_v)

            for g in range(LA):
                gather(g, g % NB)

            def step(t, c2):
                bt = lax.rem(t, NB)
                for k in range(NB):
                    @pl.when(bt == k)
                    def _(k=k):
                        gather_wait(k)
                        scatter(t, k)

                @pl.when(t + LA < SCHUNK)
                def _():
                    bp = lax.rem(t + LA, NB)
                    for k in range(NB):
                        @pl.when(bp == k)
                        def _(k=k):
                            @pl.when(t >= 1)
                            def _():
                                scatter_wait(k)
                            gather(t + LA, k)

                return c2

            lax.fori_loop(0, SCHUNK, step, 0)
            for k in range(NB):
                scatter_wait(k)
            return carry

        lax.fori_loop(0, SUP, sup_body, 0)
        plsc.subcore_barrier()
        pltpu.sync_copy(h_sh.at[pl.ds(base, ROWS_PER_TILE)],
                        out_hbm.at[c, pl.ds(base, ROWS_PER_TILE)])

    return seg_sum(src_r, dst_r, entity_embed, zrows)


def _tc_dense(hp, entity_embed, w1t, b1, w2t, b2):
    rows = 2000
    grid = N_NODES // rows

    def body(hpb, e, w1, bb1, w2, bb2, o):
        h = hpb[0] + hpb[1]
        a = jnp.dot(h, w1[...], preferred_element_type=jnp.float32) + bb1[...]
        b = jnp.dot(e[...] * h, w2[...], preferred_element_type=jnp.float32) + bb2[...]
        o[...] = jnp.where(a >= 0, a, 0.01 * a) + jnp.where(b >= 0, b, 0.01 * b)

    blk = pl.BlockSpec((rows, DIM), lambda i: (i, 0))
    wblk = pl.BlockSpec((DIM, DIM), lambda i: (0, 0))
    bblk = pl.BlockSpec((1, DIM), lambda i: (0, 0))
    return pl.pallas_call(
        body,
        grid=(grid,),
        in_specs=[pl.BlockSpec((NC, rows, DIM), lambda i: (0, i, 0)),
                  blk, wblk, bblk, wblk, bblk],
        out_specs=blk,
        out_shape=jax.ShapeDtypeStruct((N_NODES, DIM), jnp.float32),
    )(hp, entity_embed, w1t, b1, w2t, b2)


def kernel(mode, edge_index, entity_embed, W1, b1, W2, b2):
    src_r = edge_index[0].reshape(NC, NS, SUP, SCHUNK, CHUNK)
    dst_r = edge_index[1].reshape(NC, NS, SUP, SCHUNK, CHUNK)
    zrows = jnp.zeros((ZROWS, DIM), jnp.float32)
    hp = _sc_segment_sum(src_r, dst_r, entity_embed, zrows)
    return _tc_dense(hp, entity_embed,
                     W1.T, b1.reshape(1, DIM), W2.T, b2.reshape(1, DIM))


# speedup vs baseline: 1.0296x; 1.0296x over previous
import functools

import jax
import jax.numpy as jnp
from jax import lax
from jax.experimental import pallas as pl
from jax.experimental.pallas import tpu as pltpu
from jax.experimental.pallas import tpu_sc as plsc

N_NODES = 10000
N_EDGES = 320000
DIM = 128

NC = 2
NS = 16
EDGES_PER_TILE = N_EDGES // (NC * NS)
CHUNK = 80
SUP = 5
SCHUNK = 25
NCHUNK = SUP * SCHUNK
LGRP = CHUNK // 16
NB = 3
LA = NB - 1
N_PAD = 10240
ROWS_PER_TILE = N_PAD // NS
ZROWS = 64
SHIFT = 14
MASK = (1 << SHIFT) - 1


def _sc_segment_sum(src_r, dst_r, entity_embed, zrows):
    mesh = plsc.VectorSubcoreMesh(core_axis_name="c", subcore_axis_name="s")

    @functools.partial(
        pl.kernel,
        mesh=mesh,
        out_type=jax.ShapeDtypeStruct((NC, N_PAD, DIM), jnp.float32),
        scratch_types=[
            pltpu.VMEM((2, SCHUNK, CHUNK), jnp.int32),
            pltpu.VMEM((2, SCHUNK, CHUNK), jnp.int32),
            pltpu.VMEM((NB, CHUNK, DIM), jnp.float32),
            pltpu.VMEM_SHARED((N_PAD, DIM), jnp.float32),
        ] + [pltpu.SemaphoreType.DMA] * (2 * NB + 1),
    )
    def seg_sum(src_hbm, dst_hbm, ent_hbm, z_hbm, out_hbm,
                src_v, dst_v, rows_v, h_sh, *sems):
        gsem = sems[:NB]
        ssem = sems[NB:2 * NB]
        psem = sems[2 * NB]
        c = lax.axis_index("c")
        s = lax.axis_index("s")
        base = s * ROWS_PER_TILE
        for z in range(ROWS_PER_TILE // ZROWS):
            pltpu.async_copy(z_hbm, h_sh.at[pl.ds(base + z * ZROWS, ZROWS)],
                             gsem[0])
        for z in range(ROWS_PER_TILE // ZROWS):
            pltpu.make_async_copy(z_hbm, h_sh.at[pl.ds(base, ZROWS)],
                                  gsem[0]).wait()

        plsc.subcore_barrier()

        def gather(par, g, b):
            pltpu.async_copy(ent_hbm.at[src_v.at[par, g]], rows_v.at[b],
                             gsem[b])

        def gather_wait(b):
            pltpu.make_async_copy(ent_hbm.at[src_v.at[0, 0]], rows_v.at[b],
                                  gsem[b]).wait()

        def scatter(par, g, b):
            pltpu.async_copy(rows_v.at[b], h_sh.at[dst_v.at[par, g]], ssem[b],
                             add=True)

        def scatter_wait(b):
            pltpu.make_async_copy(rows_v.at[b], h_sh.at[dst_v.at[0, 0]],
                                  ssem[b]).wait()

        pltpu.sync_copy(src_hbm.at[c, s, 0], src_v.at[0])
        pltpu.sync_copy(dst_hbm.at[c, s, 0], dst_v.at[0])

        def sup_body(q, carry):
            par = lax.rem(q, 2)
            parn = lax.rem(q + 1, 2)

            @pl.when(q + 1 < SUP)
            def _():
                pltpu.async_copy(src_hbm.at[c, s, q + 1], src_v.at[parn],
                                 psem)
                pltpu.async_copy(dst_hbm.at[c, s, q + 1], dst_v.at[parn],
                                 psem)

            for g in range(LA):
                gather(par, g, g % NB)

            def step(t, c2):
                bt = lax.rem(t, NB)
                for k in range(NB):
                    @pl.when(bt == k)
                    def _(k=k):
                        gather_wait(k)
                        scatter(par, t, k)

                @pl.when(t + LA < SCHUNK)
                def _():
                    bp = lax.rem(t + LA, NB)
                    for k in range(NB):
                        @pl.when(bp == k)
                        def _(k=k):
                            @pl.when(t >= 1)
                            def _():
                                scatter_wait(k)
                            gather(par, t + LA, k)

                return c2

            lax.fori_loop(0, SCHUNK, step, 0)
            for k in range(NB):
                scatter_wait(k)

            @pl.when(q + 1 < SUP)
            def _():
                pltpu.make_async_copy(src_hbm.at[c, s, 0], src_v.at[0],
                                      psem).wait()
                pltpu.make_async_copy(dst_hbm.at[c, s, 0], dst_v.at[0],
                                      psem).wait()

            return carry

        lax.fori_loop(0, SUP, sup_body, 0)
        plsc.subcore_barrier()
        pltpu.sync_copy(h_sh.at[pl.ds(base, ROWS_PER_TILE)],
                        out_hbm.at[c, pl.ds(base, ROWS_PER_TILE)])

    return seg_sum(src_r, dst_r, entity_embed, zrows)


def _tc_dense(hp, entity_embed, w1t, b1, w2t, b2):
    rows = 2000
    grid = N_NODES // rows

    def body(hpb, e, w1, bb1, w2, bb2, o):
        h = hpb[0] + hpb[1]
        a = jnp.dot(h, w1[...], preferred_element_type=jnp.float32) + bb1[...]
        b = jnp.dot(e[...] * h, w2[...], preferred_element_type=jnp.float32) + bb2[...]
        o[...] = jnp.where(a >= 0, a, 0.01 * a) + jnp.where(b >= 0, b, 0.01 * b)

    blk = pl.BlockSpec((rows, DIM), lambda i: (i, 0))
    wblk = pl.BlockSpec((DIM, DIM), lambda i: (0, 0))
    bblk = pl.BlockSpec((1, DIM), lambda i: (0, 0))
    return pl.pallas_call(
        body,
        grid=(grid,),
        in_specs=[pl.BlockSpec((NC, rows, DIM), lambda i: (0, i, 0)),
                  blk, wblk, bblk, wblk, bblk],
        out_specs=blk,
        out_shape=jax.ShapeDtypeStruct((N_NODES, DIM), jnp.float32),
    )(hp, entity_embed, w1t, b1, w2t, b2)


def kernel(mode, edge_index, entity_embed, W1, b1, W2, b2):
    src_r = edge_index[0].reshape(NC, NS, SUP, SCHUNK, CHUNK)
    dst_r = edge_index[1].reshape(NC, NS, SUP, SCHUNK, CHUNK)
    zrows = jnp.zeros((ZROWS, DIM), jnp.float32)
    hp = _sc_segment_sum(src_r, dst_r, entity_embed, zrows)
    return _tc_dense(hp, entity_embed,
                     W1.T, b1.reshape(1, DIM), W2.T, b2.reshape(1, DIM))


# speedup vs baseline: 1.0337x; 1.0040x over previous
import functools

import jax
import jax.numpy as jnp
from jax import lax
from jax.experimental import pallas as pl
from jax.experimental.pallas import tpu as pltpu
from jax.experimental.pallas import tpu_sc as plsc

N_NODES = 10000
N_EDGES = 320000
DIM = 128

NC = 2
NS = 16
EDGES_PER_TILE = N_EDGES // (NC * NS)
CHUNK = 80
SUP = 5
SCHUNK = 25
NCHUNK = SUP * SCHUNK
LGRP = CHUNK // 16
NB = 3
LA = NB - 1
N_PAD = 10240
ROWS_PER_TILE = N_PAD // NS
ZROWS = 64
SHIFT = 14
MASK = (1 << SHIFT) - 1


def _sc_segment_sum(src_r, dst_r, entity_embed, zrows):
    mesh = plsc.VectorSubcoreMesh(core_axis_name="c", subcore_axis_name="s")

    @functools.partial(
        pl.kernel,
        mesh=mesh,
        out_type=jax.ShapeDtypeStruct((NC, N_PAD, DIM), jnp.float32),
        scratch_types=[
            pltpu.VMEM((2, SCHUNK, CHUNK), jnp.int32),
            pltpu.VMEM((2, SCHUNK, CHUNK), jnp.int32),
            pltpu.VMEM((NB, CHUNK, DIM), jnp.float32),
            pltpu.VMEM_SHARED((N_PAD, DIM), jnp.float32),
        ] + [pltpu.SemaphoreType.DMA] * (2 * NB + 1),
    )
    def seg_sum(src_hbm, dst_hbm, ent_hbm, z_hbm, out_hbm,
                src_v, dst_v, rows_v, h_sh, *sems):
        gsem = sems[:NB]
        ssem = sems[NB:2 * NB]
        psem = sems[2 * NB]
        c = lax.axis_index("c")
        s = lax.axis_index("s")
        base = s * ROWS_PER_TILE
        for z in range(ROWS_PER_TILE // ZROWS):
            pltpu.async_copy(z_hbm, h_sh.at[pl.ds(base + z * ZROWS, ZROWS)],
                             gsem[0])
        for z in range(ROWS_PER_TILE // ZROWS):
            pltpu.make_async_copy(z_hbm, h_sh.at[pl.ds(base, ZROWS)],
                                  gsem[0]).wait()

        plsc.subcore_barrier()

        def gather(par, g, b):
            pltpu.async_copy(ent_hbm.at[src_v.at[par, g]], rows_v.at[b],
                             gsem[b])

        def gather_wait(b):
            pltpu.make_async_copy(ent_hbm.at[src_v.at[0, 0]], rows_v.at[b],
                                  gsem[b]).wait()

        def scatter(par, g, b):
            pltpu.async_copy(rows_v.at[b], h_sh.at[dst_v.at[par, g]], ssem[b],
                             add=True)

        def scatter_wait(b):
            pltpu.make_async_copy(rows_v.at[b], h_sh.at[dst_v.at[0, 0]],
                                  ssem[b]).wait()

        pltpu.sync_copy(src_hbm.at[c, s, 0], src_v.at[0])
        pltpu.sync_copy(dst_hbm.at[c, s, 0], dst_v.at[0])

        def sup_body(q, carry):
            par = lax.rem(q, 2)
            parn = lax.rem(q + 1, 2)

            @pl.when(q + 1 < SUP)
            def _():
                pltpu.async_copy(src_hbm.at[c, s, q + 1], src_v.at[parn],
                                 psem)
                pltpu.async_copy(dst_hbm.at[c, s, q + 1], dst_v.at[parn],
                                 psem)

            for g in range(LA):
                gather(par, g, g % NB)

            def step(t, c2):
                bt = lax.rem(t, NB)
                for k in range(NB):
                    @pl.when(bt == k)
                    def _(k=k):
                        gather_wait(k)
                        scatter(par, t, k)

                @pl.when(t + LA < SCHUNK)
                def _():
                    bp = lax.rem(t + LA, NB)
                    for k in range(NB):
                        @pl.when(bp == k)
                        def _(k=k):
                            @pl.when(t >= 1)
                            def _():
                                scatter_wait(k)
                            gather(par, t + LA, k)

                return c2

            lax.fori_loop(0, SCHUNK, step, 0)
            for k in range(NB):
                scatter_wait(k)

            @pl.when(q + 1 < SUP)
            def _():
                pltpu.make_async_copy(src_hbm.at[c, s, 0], src_v.at[0],
                                      psem).wait()
                pltpu.make_async_copy(dst_hbm.at[c, s, 0], dst_v.at[0],
                                      psem).wait()

            return carry

        lax.fori_loop(0, SUP, sup_body, 0)
        plsc.subcore_barrier()
        pltpu.sync_copy(h_sh.at[pl.ds(base, ROWS_PER_TILE)],
                        out_hbm.at[c, pl.ds(base, ROWS_PER_TILE)])

    return seg_sum(src_r, dst_r, entity_embed, zrows)


def _tc_dense(hp, entity_embed, w1t, b1, w2t, b2):
    rows = 2000
    grid = N_NODES // rows

    dn = (((1,), (1,)), ((), ()))

    def body(hpb, e, w1, bb1, w2, bb2, o):
        h = hpb[0] + hpb[1]
        a = lax.dot_general(h, w1[...], dn,
                            preferred_element_type=jnp.float32) + bb1[...]
        b = lax.dot_general(e[...] * h, w2[...], dn,
                            preferred_element_type=jnp.float32) + bb2[...]
        o[...] = jnp.where(a >= 0, a, 0.01 * a) + jnp.where(b >= 0, b, 0.01 * b)

    blk = pl.BlockSpec((rows, DIM), lambda i: (i, 0))
    wblk = pl.BlockSpec((DIM, DIM), lambda i: (0, 0))
    bblk = pl.BlockSpec((1, DIM), lambda i: (0, 0))
    return pl.pallas_call(
        body,
        grid=(grid,),
        in_specs=[pl.BlockSpec((NC, rows, DIM), lambda i: (0, i, 0)),
                  blk, wblk, bblk, wblk, bblk],
        out_specs=blk,
        out_shape=jax.ShapeDtypeStruct((N_NODES, DIM), jnp.float32),
    )(hp, entity_embed, w1t, b1, w2t, b2)


def kernel(mode, edge_index, entity_embed, W1, b1, W2, b2):
    src_r = edge_index[0].reshape(NC, NS, SUP, SCHUNK, CHUNK)
    dst_r = edge_index[1].reshape(NC, NS, SUP, SCHUNK, CHUNK)
    zrows = jnp.zeros((ZROWS, DIM), jnp.float32)
    hp = _sc_segment_sum(src_r, dst_r, entity_embed, zrows)
    return _tc_dense(hp, entity_embed,
                     W1, b1.reshape(1, DIM), W2, b2.reshape(1, DIM))
